# baseline (speedup 1.0000x reference)
import functools

import jax
import jax.numpy as jnp
from jax import lax
from jax.experimental import pallas as pl
from jax.experimental.pallas import tpu as pltpu
from jax.experimental.pallas import tpu_sc as plsc

_R = 4
_CH = 128
_BE = 4000


def _mlp_body(ff_ref, ev_ref, dst_ref, w0_ref, b0_ref, w1r_ref, b1_ref,
              out_ref, eidx_ref):
    x = ff_ref[...]
    h = jnp.dot(x, w0_ref[...], preferred_element_type=jnp.float32)
    h = h + b0_ref[...]
    h = h * jax.nn.sigmoid(h)
    scale = jnp.sum(h * w1r_ref[...], axis=1, keepdims=True) + b1_ref[...]
    out_ref[...] = scale * ev_ref[...]
    lane = jax.lax.broadcasted_iota(jnp.int32, (ff_ref.shape[0], _R), 1)
    eidx_ref[...] = dst_ref[...] * _R + lane


def _mlp_call(ff, ev4, dst2, w0, b0r, w1r, b1r):
    e, d = ff.shape
    grid = e // _BE
    return pl.pallas_call(
        _mlp_body,
        grid=(grid,),
        in_specs=[
            pl.BlockSpec((_BE, d), lambda i: (i, 0)),
            pl.BlockSpec((_BE, _R), lambda i: (i, 0)),
            pl.BlockSpec((_BE, 1), lambda i: (i, 0)),
            pl.BlockSpec((d, d), lambda i: (0, 0)),
            pl.BlockSpec((1, d), lambda i: (0, 0)),
            pl.BlockSpec((1, d), lambda i: (0, 0)),
            pl.BlockSpec((1, 1), lambda i: (0, 0)),
        ],
        out_specs=[
            pl.BlockSpec((_BE, _R), lambda i: (i, 0)),
            pl.BlockSpec((_BE, _R), lambda i: (i, 0)),
        ],
        out_shape=[
            jax.ShapeDtypeStruct((e, _R), jnp.float32),
            jax.ShapeDtypeStruct((e, _R), jnp.int32),
        ],
    )(ff, ev4, dst2, w0, b0r, w1r, b1r)


def _make_sc_scatter(n_elem, acc_elems):
    n_chunks = n_elem // _CH
    base = n_chunks // 32
    rem = n_chunks % 32
    per_s = acc_elems // 16
    mesh = plsc.VectorSubcoreMesh(core_axis_name="c", subcore_axis_name="s")

    @functools.partial(
        pl.kernel,
        out_type=jax.ShapeDtypeStruct((2, acc_elems), jnp.float32),
        mesh=mesh,
        scratch_types=[
            pltpu.VMEM((_CH,), jnp.int32),
            pltpu.VMEM((_CH,), jnp.float32),
            pltpu.VMEM_SHARED((acc_elems,), jnp.float32),
            pltpu.SemaphoreType.DMA,
        ],
    )
    def sc_scatter(vals_hbm, eidx_hbm, zeros_hbm, out_hbm,
                   idx_v, vals_v, acc_sh, sem):
        c = lax.axis_index("c")
        s = lax.axis_index("s")
        wid = s * 2 + c
        pltpu.sync_copy(zeros_hbm.at[pl.ds(s * per_s, per_s)],
                        acc_sh.at[pl.ds(s * per_s, per_s)])
        plsc.subcore_barrier()
        start = wid * base + jnp.minimum(wid, rem)
        cnt = base + jnp.where(wid < rem, 1, 0)

        def body(j, carry):
            e0 = (start + j) * _CH
            pltpu.sync_copy(eidx_hbm.at[pl.ds(e0, _CH)], idx_v)
            pltpu.sync_copy(vals_hbm.at[pl.ds(e0, _CH)], vals_v)
            pltpu.async_copy(vals_v, acc_sh.at[idx_v], sem, add=True).wait()
            return carry

        lax.fori_loop(0, cnt, body, 0)
        plsc.subcore_barrier()
        pltpu.sync_copy(acc_sh.at[pl.ds(s * per_s, per_s)],
                        out_hbm.at[c].at[pl.ds(s * per_s, per_s)])

    return sc_scatter


def _combine_body(a_ref, b_ref, out_ref):
    out_ref[...] = a_ref[...] + b_ref[...]


def _combine_call(a, b):
    n, r = a.shape
    return pl.pallas_call(
        _combine_body,
        out_shape=jax.ShapeDtypeStruct((n, r), jnp.float32),
    )(a, b)


def kernel(force_features, edge_vectors, pos, edge_index_dst, W0, b0, W1, b1):
    e, d = force_features.shape
    n = pos.shape[0]
    n_pad = ((n + 255) // 256) * 256

    ev4 = jnp.pad(edge_vectors, ((0, 0), (0, _R - edge_vectors.shape[1])))
    dst2 = edge_index_dst.reshape(e, 1)
    b0r = b0.reshape(1, d)
    w1r = W1.reshape(1, d)
    b1r = b1.reshape(1, 1)

    forces4, eidx = _mlp_call(force_features, ev4, dst2, W0, b0r, w1r, b1r)

    acc_elems = n_pad * _R
    zeros = jnp.zeros((acc_elems,), dtype=jnp.float32)
    partials = _make_sc_scatter(e * _R, acc_elems)(
        forces4.reshape(e * _R), eidx.reshape(e * _R), zeros)

    out4 = _combine_call(partials[0].reshape(n_pad, _R),
                         partials[1].reshape(n_pad, _R))
    return out4[:n, :3]


# speedup vs baseline: 2.8733x; 2.8733x over previous
import functools

import jax
import jax.numpy as jnp
from jax import lax
from jax.experimental import pallas as pl
from jax.experimental.pallas import tpu as pltpu
from jax.experimental.pallas import tpu_sc as plsc

_BR = 25
_K = 8


def _mlp_body(ff_ref, evx_ref, evy_ref, evz_ref, w0_ref, b0_ref, w1r_ref,
              b1_ref, vx_ref, vy_ref, vz_ref):
    x = ff_ref[...]
    h = jnp.dot(x, w0_ref[...], preferred_element_type=jnp.float32)
    h = h + b0_ref[...]
    h = h * jax.nn.sigmoid(h)
    scale = jnp.sum(h * w1r_ref[...], axis=1) + b1_ref[0, 0]
    s2 = scale.reshape(_BR, 128)
    vx_ref[0] = s2 * evx_ref[0]
    vy_ref[0] = s2 * evy_ref[0]
    vz_ref[0] = s2 * evz_ref[0]


def _mlp_call(ff, evx, evy, evz, w0, b0r, w1r, b1r):
    e, d = ff.shape
    grid = e // (128 * _BR)
    be = _BR * 128
    ev_spec = pl.BlockSpec((1, _BR, 128), lambda i: (i, 0, 0))
    out_sds = jax.ShapeDtypeStruct((grid, _BR, 128), jnp.float32)
    return pl.pallas_call(
        _mlp_body,
        grid=(grid,),
        in_specs=[
            pl.BlockSpec((be, d), lambda i: (i, 0)),
            ev_spec, ev_spec, ev_spec,
            pl.BlockSpec((d, d), lambda i: (0, 0)),
            pl.BlockSpec((1, d), lambda i: (0, 0)),
            pl.BlockSpec((1, d), lambda i: (0, 0)),
            pl.BlockSpec((1, 1), lambda i: (0, 0)),
        ],
        out_specs=[ev_spec, ev_spec, ev_spec],
        out_shape=[out_sds, out_sds, out_sds],
    )(ff, evx, evy, evz, w0, b0r, w1r, b1r)


def _make_sc_scatter(rows_pad, n_pad):
    rows_per_tile = rows_pad // 32
    n_batches = rows_per_tile // _K
    per_s = n_pad // 16
    mesh = plsc.VectorSubcoreMesh(core_axis_name="c", subcore_axis_name="s")

    @functools.partial(
        pl.kernel,
        out_type=jax.ShapeDtypeStruct((6, n_pad), jnp.float32),
        mesh=mesh,
        scratch_types=[
            pltpu.VMEM((_K, 128), jnp.int32),
            pltpu.VMEM((_K, 128), jnp.float32),
            pltpu.VMEM((_K, 128), jnp.float32),
            pltpu.VMEM((_K, 128), jnp.float32),
            pltpu.VMEM_SHARED((n_pad,), jnp.float32),
            pltpu.VMEM_SHARED((n_pad,), jnp.float32),
            pltpu.VMEM_SHARED((n_pad,), jnp.float32),
            pltpu.SemaphoreType.DMA,
        ],
    )
    def sc_scatter(vx_hbm, vy_hbm, vz_hbm, dst_hbm, zeros_hbm, out_hbm,
                   idx2, vx2, vy2, vz2, acc_x, acc_y, acc_z, sem):
        c = lax.axis_index("c")
        s = lax.axis_index("s")
        wid = s * 2 + c
        sl = pl.ds(s * per_s, per_s)
        pltpu.sync_copy(zeros_hbm.at[sl], acc_x.at[sl])
        pltpu.sync_copy(zeros_hbm.at[sl], acc_y.at[sl])
        pltpu.sync_copy(zeros_hbm.at[sl], acc_z.at[sl])
        plsc.subcore_barrier()
        start_row = wid * rows_per_tile

        def body(b, carry):
            r0 = start_row + b * _K
            pltpu.sync_copy(dst_hbm.at[pl.ds(r0, _K)], idx2)
            pltpu.sync_copy(vx_hbm.at[pl.ds(r0, _K)], vx2)
            pltpu.sync_copy(vy_hbm.at[pl.ds(r0, _K)], vy2)
            pltpu.sync_copy(vz_hbm.at[pl.ds(r0, _K)], vz2)
            descs = []
            for j in range(_K):
                row_idx = idx2.at[j]
                descs.append(
                    pltpu.async_copy(vx2.at[j], acc_x.at[row_idx], sem, add=True))
                descs.append(
                    pltpu.async_copy(vy2.at[j], acc_y.at[row_idx], sem, add=True))
                descs.append(
                    pltpu.async_copy(vz2.at[j], acc_z.at[row_idx], sem, add=True))
            for dsc in descs:
                dsc.wait()
            return carry

        lax.fori_loop(0, n_batches, body, 0)
        plsc.subcore_barrier()
        pltpu.sync_copy(acc_x.at[sl], out_hbm.at[c * 3 + 0, sl])
        pltpu.sync_copy(acc_y.at[sl], out_hbm.at[c * 3 + 1, sl])
        pltpu.sync_copy(acc_z.at[sl], out_hbm.at[c * 3 + 2, sl])

    return sc_scatter


def _combine_body(a_ref, b_ref, out_ref):
    out_ref[...] = a_ref[...] + b_ref[...]


def _combine_call(a, b):
    return pl.pallas_call(
        _combine_body,
        out_shape=jax.ShapeDtypeStruct(a.shape, jnp.float32),
    )(a, b)


def kernel(force_features, edge_vectors, pos, edge_index_dst, W0, b0, W1, b1):
    e, d = force_features.shape
    n = pos.shape[0]
    n_pad = ((n + 255) // 256) * 256
    rows = e // 128
    rows_pad = ((rows + 32 * _K - 1) // (32 * _K)) * (32 * _K)

    grid = e // (128 * _BR)
    evx = edge_vectors[:, 0].reshape(grid, _BR, 128)
    evy = edge_vectors[:, 1].reshape(grid, _BR, 128)
    evz = edge_vectors[:, 2].reshape(grid, _BR, 128)
    b0r = b0.reshape(1, d)
    w1r = W1.reshape(1, d)
    b1r = b1.reshape(1, 1)

    vx, vy, vz = _mlp_call(force_features, evx, evy, evz, W0, b0r, w1r, b1r)
    vx = vx.reshape(rows, 128)
    vy = vy.reshape(rows, 128)
    vz = vz.reshape(rows, 128)

    pad_rows = rows_pad - rows
    padv = ((0, pad_rows), (0, 0))
    vx = jnp.pad(vx, padv)
    vy = jnp.pad(vy, padv)
    vz = jnp.pad(vz, padv)
    pad_idx = (jnp.arange(pad_rows * 128, dtype=jnp.int32) % n).reshape(
        pad_rows, 128)
    dst2d = jnp.concatenate(
        [edge_index_dst.reshape(rows, 128), pad_idx], axis=0)

    zeros = jnp.zeros((n_pad,), dtype=jnp.float32)
    partials = _make_sc_scatter(rows_pad, n_pad)(vx, vy, vz, dst2d, zeros)

    out = _combine_call(partials[0:3], partials[3:6])
    return out.T[:n, :]


# speedup vs baseline: 7.0371x; 2.4491x over previous
import functools

import jax
import jax.numpy as jnp
from jax import lax
from jax.experimental import pallas as pl
from jax.experimental.pallas import tpu as pltpu
from jax.experimental.pallas import tpu_sc as plsc

_BR = 50
_K = 8


def _mlp_body(ff_ref, evx_ref, evy_ref, evz_ref, w0_ref, b0_ref, w1c_ref,
              b1_ref, vx_ref, vy_ref, vz_ref):
    x = ff_ref[...]
    ht = jax.lax.dot_general(w0_ref[...], x, (((0,), (1,)), ((), ())),
                             preferred_element_type=jnp.float32)
    ht = ht + b0_ref[...].reshape(128, 1)
    ht = ht * (1.0 / (1.0 + jnp.exp(-ht)))
    st = jax.lax.dot_general(w1c_ref[...], ht, (((0,), (0,)), ((), ())),
                             preferred_element_type=jnp.float32)
    s2 = (st + b1_ref[0, 0]).reshape(_BR, 128)
    vx_ref[0] = s2 * evx_ref[0]
    vy_ref[0] = s2 * evy_ref[0]
    vz_ref[0] = s2 * evz_ref[0]


def _mlp_call(ff, evx, evy, evz, w0, b0r, w1c, b1r, off, grid_h):
    e, d = ff.shape
    be = _BR * 128
    ev_spec = pl.BlockSpec((1, _BR, 128), lambda i: (i, 0, 0))
    out_sds = jax.ShapeDtypeStruct((grid_h, _BR, 128), jnp.float32)
    return pl.pallas_call(
        _mlp_body,
        grid=(grid_h,),
        in_specs=[
            pl.BlockSpec((be, d), lambda i: (i + off, 0)),
            ev_spec, ev_spec, ev_spec,
            pl.BlockSpec((d, d), lambda i: (0, 0)),
            pl.BlockSpec((1, d), lambda i: (0, 0)),
            pl.BlockSpec((d, 1), lambda i: (0, 0)),
            pl.BlockSpec((1, 1), lambda i: (0, 0)),
        ],
        out_specs=[ev_spec, ev_spec, ev_spec],
        out_shape=[out_sds, out_sds, out_sds],
    )(ff, evx, evy, evz, w0, b0r, w1c, b1r)


def _make_sc_scatter(rows_pad, n_pad):
    rows_per_tile = rows_pad // 32
    n_batches = rows_per_tile // _K
    per_s = n_pad // 16
    mesh = plsc.VectorSubcoreMesh(core_axis_name="c", subcore_axis_name="s")

    @functools.partial(
        pl.kernel,
        out_type=jax.ShapeDtypeStruct((6, n_pad), jnp.float32),
        mesh=mesh,
        scratch_types=[
            pltpu.VMEM((2, _K, 128), jnp.int32),
            pltpu.VMEM((2, _K, 128), jnp.float32),
            pltpu.VMEM((2, _K, 128), jnp.float32),
            pltpu.VMEM((2, _K, 128), jnp.float32),
            pltpu.VMEM_SHARED((n_pad,), jnp.float32),
            pltpu.VMEM_SHARED((n_pad,), jnp.float32),
            pltpu.VMEM_SHARED((n_pad,), jnp.float32),
            pltpu.SemaphoreType.DMA,
            pltpu.SemaphoreType.DMA,
            pltpu.SemaphoreType.DMA,
            pltpu.SemaphoreType.DMA,
        ],
    )
    def sc_scatter(vx_hbm, vy_hbm, vz_hbm, dst_hbm, zeros_hbm, out_hbm,
                   idx2, vx2, vy2, vz2, acc_x, acc_y, acc_z,
                   sem_st0, sem_st1, sem_sc0, sem_sc1):
        c = lax.axis_index("c")
        s = lax.axis_index("s")
        wid = s * 2 + c
        sl = pl.ds(s * per_s, per_s)
        pltpu.sync_copy(zeros_hbm.at[sl], acc_x.at[sl])
        pltpu.sync_copy(zeros_hbm.at[sl], acc_y.at[sl])
        pltpu.sync_copy(zeros_hbm.at[sl], acc_z.at[sl])
        plsc.subcore_barrier()
        start_row = wid * rows_per_tile
        sem_st = (sem_st0, sem_st1)
        sem_sc = (sem_sc0, sem_sc1)

        def stage(b):
            p = b % 2
            r0 = start_row + b * _K
            sem = sem_st[p]
            return [
                pltpu.async_copy(dst_hbm.at[pl.ds(r0, _K)], idx2.at[p], sem),
                pltpu.async_copy(vx_hbm.at[pl.ds(r0, _K)], vx2.at[p], sem),
                pltpu.async_copy(vy_hbm.at[pl.ds(r0, _K)], vy2.at[p], sem),
                pltpu.async_copy(vz_hbm.at[pl.ds(r0, _K)], vz2.at[p], sem),
            ]

        def fire(b):
            p = b % 2
            sem = sem_sc[p]
            descs = []
            for j in range(_K):
                row_idx = idx2.at[p, j]
                descs.append(pltpu.async_copy(
                    vx2.at[p, j], acc_x.at[row_idx], sem, add=True))
                descs.append(pltpu.async_copy(
                    vy2.at[p, j], acc_y.at[row_idx], sem, add=True))
                descs.append(pltpu.async_copy(
                    vz2.at[p, j], acc_z.at[row_idx], sem, add=True))
            return descs

        st = stage(0)
        prev = None
        for b in range(n_batches):
            for dsc in st:
                dsc.wait()
            cur = fire(b)
            if prev is not None:
                for dsc in prev:
                    dsc.wait()
            if b + 1 < n_batches:
                st = stage(b + 1)
            prev = cur
        for dsc in prev:
            dsc.wait()
        plsc.subcore_barrier()
        pltpu.sync_copy(acc_x.at[sl], out_hbm.at[c * 3 + 0, sl])
        pltpu.sync_copy(acc_y.at[sl], out_hbm.at[c * 3 + 1, sl])
        pltpu.sync_copy(acc_z.at[sl], out_hbm.at[c * 3 + 2, sl])

    return sc_scatter


def _combine_body(a_ref, b_ref, c_ref, d_ref, out_ref):
    out_ref[...] = (a_ref[...] + b_ref[...]) + (c_ref[...] + d_ref[...])


def _combine_call(a, b, c, d):
    return pl.pallas_call(
        _combine_body,
        out_shape=jax.ShapeDtypeStruct(a.shape, jnp.float32),
    )(a, b, c, d)


def _prep_half(v, rows_h, rows_pad_h):
    v = v.reshape(rows_h, 128)
    return jnp.pad(v, ((0, rows_pad_h - rows_h), (0, 0)))


def kernel(force_features, edge_vectors, pos, edge_index_dst, W0, b0, W1, b1):
    e, d = force_features.shape
    n = pos.shape[0]
    n_pad = ((n + 255) // 256) * 256
    rows = e // 128
    rows_h = rows // 2
    rows_pad_h = ((rows_h + 32 * _K - 1) // (32 * _K)) * (32 * _K)

    grid = e // (128 * _BR)
    grid_h = grid // 2
    evc = [edge_vectors[:, i].reshape(grid, _BR, 128) for i in range(3)]
    b0r = b0.reshape(1, d)
    b1r = b1.reshape(1, 1)
    dst_rows = edge_index_dst.reshape(rows, 128)
    pad_rows = rows_pad_h - rows_h
    pad_idx = (jnp.arange(pad_rows * 128, dtype=jnp.int32) % n).reshape(
        pad_rows, 128)
    zeros = jnp.zeros((n_pad,), dtype=jnp.float32)
    sc_call = _make_sc_scatter(rows_pad_h, n_pad)

    partials = []
    for half in range(2):
        evx_h = evc[0][half * grid_h:(half + 1) * grid_h]
        evy_h = evc[1][half * grid_h:(half + 1) * grid_h]
        evz_h = evc[2][half * grid_h:(half + 1) * grid_h]
        vx, vy, vz = _mlp_call(force_features, evx_h, evy_h, evz_h,
                               W0, b0r, W1, b1r, half * grid_h, grid_h)
        vx = _prep_half(vx, rows_h, rows_pad_h)
        vy = _prep_half(vy, rows_h, rows_pad_h)
        vz = _prep_half(vz, rows_h, rows_pad_h)
        dst_h = jnp.concatenate(
            [dst_rows[half * rows_h:(half + 1) * rows_h], pad_idx], axis=0)
        partials.append(sc_call(vx, vy, vz, dst_h, zeros))

    out = _combine_call(partials[0][0:3], partials[0][3:6],
                        partials[1][0:3], partials[1][3:6])
    return out.T[:n, :]
